# baseline (speedup 1.0000x reference)
import functools

import jax
import jax.numpy as jnp
from jax import lax
from jax.experimental import pallas as pl
from jax.experimental.pallas import tpu as pltpu
from jax.experimental.pallas import tpu_sc as plsc

_MESH = plsc.VectorSubcoreMesh(core_axis_name="c", subcore_axis_name="s")
_INFO = plsc.get_sparse_core_info()
_NC, _NS = _INFO.num_cores, _INFO.num_subcores
_NW = _NC * _NS


def _make_transpose_tbl(V, D, VC):
    n_chunks = V // VC
    assert V % VC == 0 and VC % 16 == 0 and D % 16 == 0
    k_max = (n_chunks + _NW - 1) // _NW

    @functools.partial(
        pl.kernel,
        out_type=jax.ShapeDtypeStruct((V, D), jnp.float32),
        mesh=_MESH,
        scratch_types=[
            pltpu.VMEM((D, VC), jnp.float32),
            pltpu.VMEM((VC, D), jnp.float32),
        ],
        compiler_params=pltpu.CompilerParams(use_tc_tiling_on_sc=False, needs_layout_passes=False),
    )
    def transpose_tbl(tt_hbm, tbl_hbm, tin, tout):
        wid = lax.axis_index("s") * _NC + lax.axis_index("c")
        lanes = lax.iota(jnp.int32, 16)

        @pl.loop(0, k_max)
        def _(k):
            c = wid + k * _NW

            @pl.when(c < n_chunks)
            def _():
                v0 = c * VC
                pltpu.sync_copy(tt_hbm.at[:, pl.ds(v0, VC)], tin)

                @pl.loop(0, VC)
                def _(v):
                    col = jnp.full((16,), v, jnp.int32)
                    for jb in range(D // 16):
                        vals = plsc.load_gather(tin, [lanes + jb * 16, col])
                        tout[v, pl.ds(jb * 16, 16)] = vals

                pltpu.sync_copy(tout, tbl_hbm.at[pl.ds(v0, VC)])

    return transpose_tbl


def _make_gather_t(S, N, V, D, C):
    n_units = S * (N // C)
    per_w = n_units // _NW
    blocks_per_s = N // C
    assert N % C == 0 and n_units % _NW == 0 and per_w % 2 == 0

    @functools.partial(
        pl.kernel,
        out_type=jax.ShapeDtypeStruct((S, D, N), jnp.float32),
        mesh=_MESH,
        scratch_types=[
            pltpu.VMEM((per_w, C), jnp.int32),
            pltpu.VMEM((2, C, D), jnp.float32),
            pltpu.VMEM((2, D, C), jnp.float32),
            pltpu.SemaphoreType.DMA((2,)),
            pltpu.SemaphoreType.DMA((2,)),
        ],
        compiler_params=pltpu.CompilerParams(use_tc_tiling_on_sc=False, needs_layout_passes=False),
    )
    def gather_t(idx_hbm, tbl_hbm, out_hbm, idx_v, rows_v, tbuf, gsem, wsem):
        wid = lax.axis_index("s") * _NC + lax.axis_index("c")
        u0 = wid * per_w
        pltpu.sync_copy(idx_hbm.at[pl.ds(u0, per_w)], idx_v)
        lanes = lax.iota(jnp.int32, 16)

        def gather_copy(lu, b):
            return pltpu.make_async_copy(
                tbl_hbm.at[idx_v.at[lu]], rows_v.at[b], gsem.at[b]
            )

        def wb_copy(lu, b):
            u = u0 + lu
            s = u // blocks_per_s
            n0 = (u % blocks_per_s) * C
            return pltpu.make_async_copy(
                tbuf.at[b], out_hbm.at[s, :, pl.ds(n0, C)], wsem.at[b]
            )

        def transpose_block(b):
            @pl.loop(0, D)
            def _(j):
                col = jnp.full((16,), j, jnp.int32)
                for rb in range(C // 16):
                    vals = plsc.load_gather(
                        rows_v.at[b], [lanes + rb * 16, col]
                    )
                    tbuf[b, j, pl.ds(rb * 16, 16)] = vals

        gather_copy(0, 0).start()

        @pl.loop(0, per_w, step=2)
        def _(lu):
            for b in range(2):
                cur = lu + b
                gather_copy(cur, b).wait()

                @pl.when(cur + 1 < per_w)
                def _():
                    gather_copy(cur + 1, 1 - b).start()

                @pl.when(cur >= 2)
                def _():
                    wb_copy(cur - 2, b).wait()

                transpose_block(b)
                wb_copy(cur, b).start()

        for b in range(2):
            wb_copy(per_w - 2 + b, b).wait()

    return gather_t


def kernel(indices, wte):
    n, s = indices.shape
    V, D = wte.shape
    C = 256
    tbl = _make_transpose_tbl(V, D, VC=400)(wte.T)
    idx2 = indices.T.reshape(s * (n // C), C)
    out_t = _make_gather_t(s, n, V, D, C)(idx2, tbl)
    return out_t.transpose(2, 0, 1)


# speedup vs baseline: 6.1098x; 6.1098x over previous
import functools

import jax
import jax.numpy as jnp
from jax import lax
from jax.experimental import pallas as pl
from jax.experimental.pallas import tpu as pltpu
from jax.experimental.pallas import tpu_sc as plsc


def _make_gather(N, S, D, NB):
    info = plsc.get_sparse_core_info()
    NC, NS = info.num_cores, info.num_subcores
    NW = NC * NS
    rows_per_w = N // NW
    n_groups = rows_per_w // NB
    assert N % NW == 0 and rows_per_w % NB == 0

    mesh = plsc.VectorSubcoreMesh(core_axis_name="c", subcore_axis_name="s")

    @functools.partial(
        pl.kernel,
        out_type=jax.ShapeDtypeStruct((N, S, D), jnp.float32),
        mesh=mesh,
        scratch_types=[
            pltpu.VMEM((rows_per_w, S), jnp.int32),
            pltpu.VMEM((NB, S, D), jnp.float32),
            pltpu.SemaphoreType.DMA((NB,)),
            pltpu.SemaphoreType.DMA((NB,)),
        ],
        compiler_params=pltpu.CompilerParams(use_tc_tiling_on_sc=False),
    )
    def gather(idx_hbm, table_hbm, out_hbm, idx_v, rows_v, gsem, wsem):
        wid = lax.axis_index("s") * NC + lax.axis_index("c")
        row0 = wid * rows_per_w
        pltpu.sync_copy(idx_hbm.at[pl.ds(row0, rows_per_w)], idx_v)

        def gather_copy(i, b):
            return pltpu.make_async_copy(
                table_hbm.at[idx_v.at[i]], rows_v.at[b], gsem.at[b]
            )

        def wb_copy(i, b):
            return pltpu.make_async_copy(
                rows_v.at[b], out_hbm.at[row0 + i], wsem.at[b]
            )

        for b in range(NB):
            gather_copy(b, b).start()

        @pl.loop(1, n_groups)
        def _(g):
            i0 = g * NB
            for b in range(NB):
                prev = i0 - NB + b
                gather_copy(prev, b).wait()
                wb_copy(prev, b).start()
            for b in range(NB):
                wb_copy(i0 - NB + b, b).wait()
                gather_copy(i0 + b, b).start()

        last0 = (n_groups - 1) * NB
        for b in range(NB):
            gather_copy(last0 + b, b).wait()
            wb_copy(last0 + b, b).start()
        for b in range(NB):
            wb_copy(last0 + b, b).wait()

    return gather


def kernel(indices, wte):
    n, s = indices.shape
    _, D = wte.shape
    gather = _make_gather(n, s, D, NB=4)
    return gather(indices, wte)
